# baseline (speedup 1.0000x reference)
import functools

import jax
import jax.numpy as jnp
from jax.experimental import pallas as pl
from jax.experimental.pallas import tpu as pltpu


def _cdiv(a, b):
    return (a + b - 1) // b


def _masked_k_dot(a, t, nw, k0):
    a_head = a[:, :k0]
    a_tail = a[:, k0:]
    col = k0 + jax.lax.broadcasted_iota(jnp.int32, a_tail.shape, 1)
    a_tail = jnp.where(col < nw, a_tail, 0.0)
    acc = jnp.dot(a_head, t[:k0, :], preferred_element_type=jnp.float32)
    acc += jnp.dot(a_tail, t[k0:, :], preferred_element_type=jnp.float32)
    return acc


def _fused_kernel(nw, k0, ta, tb, na, nb,
                  g2_ref, x_ref, p_ref, w_ref, g1_ref, b_ref,
                  o_ref, t1_ref, t2_ref):
    i = pl.program_id(0)

    @pl.when(i < na)
    def _phase_a():
        gx = jnp.dot(g2_ref[...], x_ref[...],
                     preferred_element_type=jnp.float32)
        acc = jnp.dot(gx, p_ref[...], preferred_element_type=jnp.float32)
        row = i * ta + jax.lax.broadcasted_iota(jnp.int32, acc.shape, 0)
        t1_ref[pl.ds(i * ta, ta), :] = jnp.where(row < nw, acc, 0.0)

    @pl.when(jnp.logical_and(i >= na, i < na + nb))
    def _phase_b():
        j = i - na
        acc = _masked_k_dot(w_ref[...], t1_ref[...], nw, k0)
        row = j * tb + jax.lax.broadcasted_iota(jnp.int32, acc.shape, 0)
        t2_ref[pl.ds(j * tb, tb), :] = jnp.where(row < nw, acc, 0.0)

    @pl.when(i >= na + nb)
    def _phase_c():
        acc = _masked_k_dot(g1_ref[...], t2_ref[...], nw, k0)
        o_ref[...] = acc + b_ref[...]


def kernel(g1, g2, x, weight, p, bias):
    m, nw = g1.shape
    in_dim = x.shape[1]
    out_dim = p.shape[1]

    ta = 512
    tb = 256
    nwp = _cdiv(nw, 512) * 512
    k0 = (nw // 256) * 256
    na = nwp // ta
    nb = nwp // tb
    nc = m // tb

    def resident(shape):
        return pl.BlockSpec(shape, lambda i: (0, 0))

    out = pl.pallas_call(
        functools.partial(_fused_kernel, nw, k0, ta, tb, na, nb),
        out_shape=jax.ShapeDtypeStruct((m, out_dim), jnp.float32),
        grid=(na + nb + nc,),
        in_specs=[
            pl.BlockSpec((ta, m), lambda i: (jnp.minimum(i, na - 1), 0)),
            resident((m, in_dim)),
            resident((in_dim, out_dim)),
            pl.BlockSpec((tb, nwp),
                         lambda i: (jnp.clip(i - na, 0, nb - 1), 0)),
            pl.BlockSpec((tb, nwp),
                         lambda i: (jnp.clip(i - na - nb, 0, nc - 1), 0)),
            resident((1, out_dim)),
        ],
        out_specs=pl.BlockSpec(
            (tb, out_dim), lambda i: (jnp.clip(i - na - nb, 0, nc - 1), 0)),
        scratch_shapes=[
            pltpu.VMEM((nwp, out_dim), jnp.float32),
            pltpu.VMEM((nwp, out_dim), jnp.float32),
        ],
        compiler_params=pltpu.CompilerParams(
            dimension_semantics=("arbitrary",)),
    )(g2, x, p, weight, g1, bias.reshape(1, out_dim))

    return out
